# baseline (speedup 1.0000x reference)
import functools

import jax
import jax.numpy as jnp
from jax import lax
from jax.experimental import pallas as pl
from jax.experimental.pallas import tpu as pltpu
from jax.experimental.pallas import tpu_sc as plsc

N_NODES = 10000
N_EDGES = 160000
HIDDEN = 256
QF = 64
NQ = 4
HALF = 128
NC = 2
NS = 16
LANES = 16
E_PAD = 163840
EDGES_PER_TILE = E_PAD // NS
CHUNK = 128
NCHUNK = EDGES_PER_TILE // CHUNK
NPAD = 10240
NODES_PER_TILE = NPAD // NS
ZROWS = 64

ROW_BLK = 1000
EDGE_BLK = 2000


def _silu(x):
    return x * (1.0 / (1.0 + jnp.exp(-x)))


def _res(x, w0, b0, w1, b1):
    h = _silu(jnp.dot(x, w0, preferred_element_type=jnp.float32) + b0)
    h = _silu(jnp.dot(h, w1, preferred_element_type=jnp.float32) + b1)
    return x + h


def _emb_body(x_ref, ws_ref, bs_ref, wd_ref, bd_ref, src_ref, dst_ref):
    x = x_ref[...]
    src_ref[...] = _res(x, ws_ref[0], bs_ref[0], ws_ref[1], bs_ref[1])
    dst = _res(x, wd_ref[0], bd_ref[0], wd_ref[1], bd_ref[1])
    dst_ref[0, :, :] = dst[:, :HALF]
    dst_ref[1, :, :] = dst[:, HALF:]


def _emb_call(scalar, ws, bs, wd, bd):
    grid = (N_NODES // ROW_BLK,)
    return pl.pallas_call(
        _emb_body,
        grid=grid,
        in_specs=[
            pl.BlockSpec((ROW_BLK, HIDDEN), lambda r: (r, 0)),
            pl.BlockSpec((2, HIDDEN, HIDDEN), lambda r: (0, 0, 0)),
            pl.BlockSpec((2, 1, HIDDEN), lambda r: (0, 0, 0)),
            pl.BlockSpec((2, HIDDEN, HIDDEN), lambda r: (0, 0, 0)),
            pl.BlockSpec((2, 1, HIDDEN), lambda r: (0, 0, 0)),
        ],
        out_specs=[
            pl.BlockSpec((ROW_BLK, HIDDEN), lambda r: (r, 0)),
            pl.BlockSpec((2, ROW_BLK, HALF), lambda r: (0, r, 0)),
        ],
        out_shape=[
            jax.ShapeDtypeStruct((N_NODES, HIDDEN), jnp.float32),
            jax.ShapeDtypeStruct((2, N_NODES, HALF), jnp.float32),
        ],
    )(scalar, ws, bs, wd, bd)


def _filter_body(ef_ref, we_ref, cs_ref, out_ref):
    y = jnp.dot(ef_ref[...], we_ref[...], preferred_element_type=jnp.float32)
    y = y * cs_ref[...]
    out_ref[0, :, :] = y[:, :HALF]
    out_ref[1, :, :] = y[:, HALF:]


def _filter_call(ef, we, cs):
    grid = (E_PAD // EDGE_BLK,)
    return pl.pallas_call(
        _filter_body,
        grid=grid,
        in_specs=[
            pl.BlockSpec((EDGE_BLK, 16), lambda e: (e, 0)),
            pl.BlockSpec((16, HIDDEN), lambda e: (0, 0)),
            pl.BlockSpec((EDGE_BLK, 1), lambda e: (e, 0)),
        ],
        out_specs=pl.BlockSpec((2, EDGE_BLK, HALF), lambda e: (0, e, 0)),
        out_shape=jax.ShapeDtypeStruct((2, E_PAD, HALF), jnp.float32),
    )(ef, we, cs)


def _sc_edge_body(dst_hbm, filt_hbm, col_hbm, row_hbm, out_hbm,
                  cidx, ridx, gbuf, fbuf, pbuf, zbuf, wbuf, acc, sem):
    c = lax.axis_index("c")
    s = lax.axis_index("s")
    base = s * NODES_PER_TILE
    ebase = s * EDGES_PER_TILE

    def zset(i, carry):
        r = i // (QF // LANES)
        h = (i % (QF // LANES)) * LANES
        zbuf[r, pl.ds(h, LANES)] = jnp.zeros((LANES,), jnp.float32)
        return carry

    lax.fori_loop(0, (ZROWS * QF) // LANES, zset, 0)

    for qloc in range(2):
        q = c * 2 + qloc

        def zcopy(k, carry):
            pltpu.sync_copy(zbuf, acc.at[pl.ds(base + k * ZROWS, ZROWS)])
            return carry

        lax.fori_loop(0, NODES_PER_TILE // ZROWS, zcopy, 0)
        plsc.subcore_barrier()

        @pl.loop(0, NCHUNK)
        def chunk(j):
            eoff = ebase + j * CHUNK
            pltpu.sync_copy(filt_hbm.at[pl.ds(c * E_PAD + eoff, CHUNK)],
                            fbuf)
            pltpu.sync_copy(col_hbm.at[pl.ds(c * E_PAD + eoff, CHUNK)],
                            cidx)
            pltpu.async_copy(dst_hbm.at[cidx], gbuf, sem).wait()
            pltpu.sync_copy(row_hbm.at[pl.ds(eoff, CHUNK)], ridx)

            def edge(e, carry2):
                for h in range(QF // LANES):
                    sl = pl.ds(h * LANES, LANES)
                    gsl = pl.ds(qloc * QF + h * LANES, LANES)
                    pbuf[e, sl] = gbuf[e, gsl] * fbuf[e, gsl]
                return carry2

            lax.fori_loop(0, CHUNK, edge, 0)
            pltpu.sync_copy(pbuf, acc.at[ridx], add=True)

        plsc.subcore_barrier()

        def wcopy(k, carry):
            off = base + k * ZROWS
            pltpu.sync_copy(acc.at[pl.ds(off, ZROWS)], wbuf)
            pltpu.sync_copy(wbuf, out_hbm.at[c, pl.ds(off, ZROWS),
                                             pl.ds(qloc * QF, QF)])
            return carry

        lax.fori_loop(0, NODES_PER_TILE // ZROWS, wcopy, 0)


def _sc_edge_call(dst_flat, filt_flat, col_flat, row_flat):
    mesh = plsc.VectorSubcoreMesh(
        core_axis_name="c", subcore_axis_name="s",
        num_cores=NC, num_subcores=NS)
    fn = pl.kernel(
        _sc_edge_body,
        out_type=jax.ShapeDtypeStruct((2, NPAD, HALF), jnp.float32),
        mesh=mesh,
        compiler_params=pltpu.CompilerParams(use_tc_tiling_on_sc=False),
        scratch_types=[
            pltpu.VMEM((CHUNK,), jnp.int32),
            pltpu.VMEM((CHUNK,), jnp.int32),
            pltpu.VMEM((CHUNK, HALF), jnp.float32),
            pltpu.VMEM((CHUNK, HALF), jnp.float32),
            pltpu.VMEM((CHUNK, QF), jnp.float32),
            pltpu.VMEM((ZROWS, QF), jnp.float32),
            pltpu.VMEM((ZROWS, QF), jnp.float32),
            pltpu.VMEM_SHARED((NPAD, QF), jnp.float32),
            pltpu.SemaphoreType.DMA,
        ],
    )
    return fn(dst_flat, filt_flat, col_flat, row_flat)


def _post_body(conv_ref, src_ref, c_ref, cw_ref, cb_ref, ow_ref, ob_ref,
               out_ref):
    x = jnp.concatenate([conv_ref[0], conv_ref[1]], axis=1)
    x = x * c_ref[0, 0]
    x = _res(x, cw_ref[0, 0], cb_ref[0, 0], cw_ref[0, 1], cb_ref[0, 1])
    x = _res(x, cw_ref[1, 0], cb_ref[1, 0], cw_ref[1, 1], cb_ref[1, 1])
    x = x * src_ref[...]
    x = _res(x, ow_ref[0, 0], ob_ref[0, 0], ow_ref[0, 1], ob_ref[0, 1])
    x = _res(x, ow_ref[1, 0], ob_ref[1, 0], ow_ref[1, 1], ob_ref[1, 1])
    out_ref[...] = x


def _post_call(conv_split, src, c, cw, cb, ow, ob):
    grid = (N_NODES // ROW_BLK,)
    return pl.pallas_call(
        _post_body,
        grid=grid,
        in_specs=[
            pl.BlockSpec((2, ROW_BLK, HALF), lambda r: (0, r, 0)),
            pl.BlockSpec((ROW_BLK, HIDDEN), lambda r: (r, 0)),
            pl.BlockSpec((1, 1), lambda r: (0, 0)),
            pl.BlockSpec((2, 2, HIDDEN, HIDDEN), lambda r: (0, 0, 0, 0)),
            pl.BlockSpec((2, 2, 1, HIDDEN), lambda r: (0, 0, 0, 0)),
            pl.BlockSpec((2, 2, HIDDEN, HIDDEN), lambda r: (0, 0, 0, 0)),
            pl.BlockSpec((2, 2, 1, HIDDEN), lambda r: (0, 0, 0, 0)),
        ],
        out_specs=pl.BlockSpec((ROW_BLK, HIDDEN), lambda r: (r, 0)),
        out_shape=jax.ShapeDtypeStruct((N_NODES, HIDDEN), jnp.float32),
    )(conv_split, src, c, cw, cb, ow, ob)


def kernel(scalar, ef, edge_index, C, conv_smooth, emb_W, emb_b, out_W,
           out_b, conv_W, conv_b, We):
    npad_e = E_PAD - N_EDGES
    row = jnp.concatenate([
        edge_index[0].astype(jnp.int32),
        jnp.full((npad_e,), N_NODES, dtype=jnp.int32)])
    col = jnp.concatenate([
        edge_index[1].astype(jnp.int32),
        jnp.zeros((npad_e,), dtype=jnp.int32)])
    ef_p = jnp.concatenate([ef, jnp.zeros((npad_e, 16), ef.dtype)])
    cs_p = jnp.concatenate(
        [conv_smooth, jnp.zeros((npad_e, 1), conv_smooth.dtype)])

    src, dst_split = _emb_call(
        scalar,
        emb_W[0], emb_b[0].reshape(2, 1, HIDDEN),
        emb_W[1], emb_b[1].reshape(2, 1, HIDDEN))

    filt = _filter_call(ef_p, We, cs_p)

    dst_flat = dst_split.reshape(2 * N_NODES, HALF)
    offs = jnp.arange(2, dtype=jnp.int32) * N_NODES
    col_flat = (col[None, :] + offs[:, None]).reshape(2 * E_PAD)
    filt_flat = filt.reshape(2 * E_PAD, HALF)

    conv_split = _sc_edge_call(dst_flat, filt_flat, col_flat,
                               row)[:, :N_NODES, :]

    return _post_call(
        conv_split, src, C.reshape(1, 1),
        conv_W, conv_b.reshape(2, 2, 1, HIDDEN),
        out_W, out_b.reshape(2, 2, 1, HIDDEN))


# speedup vs baseline: 1.4196x; 1.4196x over previous
import functools

import jax
import jax.numpy as jnp
from jax import lax
from jax.experimental import pallas as pl
from jax.experimental.pallas import tpu as pltpu
from jax.experimental.pallas import tpu_sc as plsc

N_NODES = 10000
N_EDGES = 160000
HIDDEN = 256
QF = 64
NQ = 4
HALF = 128
NC = 2
NS = 16
LANES = 16
E_PAD = 163840
EDGES_PER_TILE = E_PAD // NS
CHUNK = 128
NCHUNK = EDGES_PER_TILE // CHUNK
NPAD = 10240
NODES_PER_TILE = NPAD // NS
ZROWS = 64

ROW_BLK = 1000
EDGE_BLK = 2000


def _silu(x):
    return x * (1.0 / (1.0 + jnp.exp(-x)))


def _res(x, w0, b0, w1, b1):
    h = _silu(jnp.dot(x, w0, preferred_element_type=jnp.float32) + b0)
    h = _silu(jnp.dot(h, w1, preferred_element_type=jnp.float32) + b1)
    return x + h


def _emb_body(x_ref, ws_ref, bs_ref, wd_ref, bd_ref, src_ref, dst_ref):
    x = x_ref[...]
    src_ref[...] = _res(x, ws_ref[0], bs_ref[0], ws_ref[1], bs_ref[1])
    dst = _res(x, wd_ref[0], bd_ref[0], wd_ref[1], bd_ref[1])
    dst_ref[0, :, :] = dst[:, :HALF]
    dst_ref[1, :, :] = dst[:, HALF:]


def _emb_call(scalar, ws, bs, wd, bd):
    grid = (N_NODES // ROW_BLK,)
    return pl.pallas_call(
        _emb_body,
        grid=grid,
        in_specs=[
            pl.BlockSpec((ROW_BLK, HIDDEN), lambda r: (r, 0)),
            pl.BlockSpec((2, HIDDEN, HIDDEN), lambda r: (0, 0, 0)),
            pl.BlockSpec((2, 1, HIDDEN), lambda r: (0, 0, 0)),
            pl.BlockSpec((2, HIDDEN, HIDDEN), lambda r: (0, 0, 0)),
            pl.BlockSpec((2, 1, HIDDEN), lambda r: (0, 0, 0)),
        ],
        out_specs=[
            pl.BlockSpec((ROW_BLK, HIDDEN), lambda r: (r, 0)),
            pl.BlockSpec((2, ROW_BLK, HALF), lambda r: (0, r, 0)),
        ],
        out_shape=[
            jax.ShapeDtypeStruct((N_NODES, HIDDEN), jnp.float32),
            jax.ShapeDtypeStruct((2, N_NODES, HALF), jnp.float32),
        ],
    )(scalar, ws, bs, wd, bd)


def _filter_body(ef_ref, we_ref, cs_ref, out_ref):
    y = jnp.dot(ef_ref[...], we_ref[...], preferred_element_type=jnp.float32)
    y = y * cs_ref[...]
    out_ref[0, :, :] = y[:, :HALF]
    out_ref[1, :, :] = y[:, HALF:]


def _filter_call(ef, we, cs):
    grid = (E_PAD // EDGE_BLK,)
    return pl.pallas_call(
        _filter_body,
        grid=grid,
        in_specs=[
            pl.BlockSpec((EDGE_BLK, 16), lambda e: (e, 0)),
            pl.BlockSpec((16, HIDDEN), lambda e: (0, 0)),
            pl.BlockSpec((EDGE_BLK, 1), lambda e: (e, 0)),
        ],
        out_specs=pl.BlockSpec((2, EDGE_BLK, HALF), lambda e: (0, e, 0)),
        out_shape=jax.ShapeDtypeStruct((2, E_PAD, HALF), jnp.float32),
    )(ef, we, cs)


def _sc_edge_body(dst_hbm, filt_hbm, col_hbm, row_hbm, out_hbm,
                  cidx, ridx, gbuf, fbuf, pbuf, zbuf, wbuf, acc,
                  gsem, fsem):
    c = lax.axis_index("c")
    s = lax.axis_index("s")
    base = s * NODES_PER_TILE
    ebase = s * EDGES_PER_TILE

    def zset(i, carry):
        r = i // (QF // LANES)
        h = (i % (QF // LANES)) * LANES
        zbuf[r, pl.ds(h, LANES)] = jnp.zeros((LANES,), jnp.float32)
        return carry

    lax.fori_loop(0, (ZROWS * QF) // LANES, zset, 0)

    for qloc in range(2):
        q = c * 2 + qloc

        def zcopy(k, carry):
            pltpu.sync_copy(zbuf, acc.at[pl.ds(base + k * ZROWS, ZROWS)])
            return carry

        lax.fori_loop(0, NODES_PER_TILE // ZROWS, zcopy, 0)
        plsc.subcore_barrier()

        def fire(j, par):
            eoff = ebase + j * CHUNK
            pltpu.sync_copy(col_hbm.at[pl.ds(c * E_PAD + eoff, CHUNK)],
                            cidx[par])
            pltpu.async_copy(filt_hbm.at[pl.ds(c * E_PAD + eoff, CHUNK)],
                             fbuf[par], fsem[par])
            pltpu.async_copy(dst_hbm.at[cidx[par]], gbuf[par], gsem[par])

        def drain(j, par):
            eoff = ebase + j * CHUNK
            pltpu.make_async_copy(
                filt_hbm.at[pl.ds(c * E_PAD + eoff, CHUNK)],
                fbuf[par], fsem[par]).wait()
            pltpu.make_async_copy(dst_hbm.at[cidx[par]], gbuf[par],
                                  gsem[par]).wait()

        fire(0, 0)

        @pl.loop(0, NCHUNK // 2)
        def chunk2(jj):
            for par in range(2):
                j = jj * 2 + par
                jn = jnp.minimum(j + 1, NCHUNK - 1)
                fire(jn, 1 - par)
                drain(j, par)
                pltpu.sync_copy(row_hbm.at[pl.ds(ebase + j * CHUNK, CHUNK)],
                                ridx)

                @pl.loop(0, CHUNK, unroll=4)
                def edge(e):
                    for h in range(QF // LANES):
                        sl = pl.ds(h * LANES, LANES)
                        gsl = pl.ds(qloc * QF + h * LANES, LANES)
                        pbuf[e, sl] = gbuf[par][e, gsl] * fbuf[par][e, gsl]

                pltpu.sync_copy(pbuf, acc.at[ridx], add=True)

        drain(NCHUNK - 1, 0)

        plsc.subcore_barrier()

        def wcopy(k, carry):
            off = base + k * ZROWS
            pltpu.sync_copy(acc.at[pl.ds(off, ZROWS)], wbuf)
            pltpu.sync_copy(wbuf, out_hbm.at[c, pl.ds(off, ZROWS),
                                             pl.ds(qloc * QF, QF)])
            return carry

        lax.fori_loop(0, NODES_PER_TILE // ZROWS, wcopy, 0)


def _sc_edge_call(dst_flat, filt_flat, col_flat, row_flat):
    mesh = plsc.VectorSubcoreMesh(
        core_axis_name="c", subcore_axis_name="s",
        num_cores=NC, num_subcores=NS)
    fn = pl.kernel(
        _sc_edge_body,
        out_type=jax.ShapeDtypeStruct((2, NPAD, HALF), jnp.float32),
        mesh=mesh,
        compiler_params=pltpu.CompilerParams(use_tc_tiling_on_sc=False),
        scratch_types=[
            [pltpu.VMEM((CHUNK,), jnp.int32)] * 2,
            pltpu.VMEM((CHUNK,), jnp.int32),
            [pltpu.VMEM((CHUNK, HALF), jnp.float32)] * 2,
            [pltpu.VMEM((CHUNK, HALF), jnp.float32)] * 2,
            pltpu.VMEM((CHUNK, QF), jnp.float32),
            pltpu.VMEM((ZROWS, QF), jnp.float32),
            pltpu.VMEM((ZROWS, QF), jnp.float32),
            pltpu.VMEM_SHARED((NPAD, QF), jnp.float32),
            [pltpu.SemaphoreType.DMA] * 2,
            [pltpu.SemaphoreType.DMA] * 2,
        ],
    )
    return fn(dst_flat, filt_flat, col_flat, row_flat)


def _post_body(conv_ref, src_ref, c_ref, cw_ref, cb_ref, ow_ref, ob_ref,
               out_ref):
    x = jnp.concatenate([conv_ref[0], conv_ref[1]], axis=1)
    x = x * c_ref[0, 0]
    x = _res(x, cw_ref[0, 0], cb_ref[0, 0], cw_ref[0, 1], cb_ref[0, 1])
    x = _res(x, cw_ref[1, 0], cb_ref[1, 0], cw_ref[1, 1], cb_ref[1, 1])
    x = x * src_ref[...]
    x = _res(x, ow_ref[0, 0], ob_ref[0, 0], ow_ref[0, 1], ob_ref[0, 1])
    x = _res(x, ow_ref[1, 0], ob_ref[1, 0], ow_ref[1, 1], ob_ref[1, 1])
    out_ref[...] = x


def _post_call(conv_split, src, c, cw, cb, ow, ob):
    grid = (N_NODES // ROW_BLK,)
    return pl.pallas_call(
        _post_body,
        grid=grid,
        in_specs=[
            pl.BlockSpec((2, ROW_BLK, HALF), lambda r: (0, r, 0)),
            pl.BlockSpec((ROW_BLK, HIDDEN), lambda r: (r, 0)),
            pl.BlockSpec((1, 1), lambda r: (0, 0)),
            pl.BlockSpec((2, 2, HIDDEN, HIDDEN), lambda r: (0, 0, 0, 0)),
            pl.BlockSpec((2, 2, 1, HIDDEN), lambda r: (0, 0, 0, 0)),
            pl.BlockSpec((2, 2, HIDDEN, HIDDEN), lambda r: (0, 0, 0, 0)),
            pl.BlockSpec((2, 2, 1, HIDDEN), lambda r: (0, 0, 0, 0)),
        ],
        out_specs=pl.BlockSpec((ROW_BLK, HIDDEN), lambda r: (r, 0)),
        out_shape=jax.ShapeDtypeStruct((N_NODES, HIDDEN), jnp.float32),
    )(conv_split, src, c, cw, cb, ow, ob)


def kernel(scalar, ef, edge_index, C, conv_smooth, emb_W, emb_b, out_W,
           out_b, conv_W, conv_b, We):
    npad_e = E_PAD - N_EDGES
    row = jnp.concatenate([
        edge_index[0].astype(jnp.int32),
        jnp.full((npad_e,), N_NODES, dtype=jnp.int32)])
    col = jnp.concatenate([
        edge_index[1].astype(jnp.int32),
        jnp.zeros((npad_e,), dtype=jnp.int32)])
    ef_p = jnp.concatenate([ef, jnp.zeros((npad_e, 16), ef.dtype)])
    cs_p = jnp.concatenate(
        [conv_smooth, jnp.zeros((npad_e, 1), conv_smooth.dtype)])

    src, dst_split = _emb_call(
        scalar,
        emb_W[0], emb_b[0].reshape(2, 1, HIDDEN),
        emb_W[1], emb_b[1].reshape(2, 1, HIDDEN))

    filt = _filter_call(ef_p, We, cs_p)

    dst_flat = dst_split.reshape(2 * N_NODES, HALF)
    offs = jnp.arange(2, dtype=jnp.int32) * N_NODES
    col_flat = (col[None, :] + offs[:, None]).reshape(2 * E_PAD)
    filt_flat = filt.reshape(2 * E_PAD, HALF)

    conv_split = _sc_edge_call(dst_flat, filt_flat, col_flat,
                               row)[:, :N_NODES, :]

    return _post_call(
        conv_split, src, C.reshape(1, 1),
        conv_W, conv_b.reshape(2, 2, 1, HIDDEN),
        out_W, out_b.reshape(2, 2, 1, HIDDEN))
